# Initial kernel scaffold; baseline (speedup 1.0000x reference)
import jax
import jax.numpy as jnp
from jax.experimental import pallas as pl


def kernel(features, edge_index, edge_type, W_rel_0, a_l_0, a_r_0, W_self_0, W_rel_1, a_l_1, a_r_1, W_self_1, W_rel_2, a_l_2, a_r_2, W_self_2):
    raise NotImplementedError("write your pallas kernel here")


# speedup vs baseline: 14.0993x; 14.0993x over previous
import functools

import jax
import jax.numpy as jnp
from jax.experimental import pallas as pl
from jax.experimental.pallas import tpu as pltpu

N = 10000
R = 20
H = 3
TN = 1000


def _trans_body(h_ref, w_ref, alt_ref, art_ref, trans_ref, lg_ref):
    t = jnp.dot(h_ref[...], w_ref[0], preferred_element_type=jnp.float32)
    trans_ref[0] = t
    el = jnp.dot(t, alt_ref[...], preferred_element_type=jnp.float32)
    er = jnp.dot(t, art_ref[...], preferred_element_type=jnp.float32)
    lg_ref[0] = jnp.concatenate([el, er], axis=1)


def _trans_call(h, W_rel, a_l, a_r):
    in_dim = h.shape[1]
    ho = W_rel.shape[2]
    out = ho // H
    heads = jnp.arange(ho) // out
    offs = jnp.arange(ho) % out
    cols = jnp.arange(8)[None, :]
    alt = jnp.where(cols == heads[:, None], a_l[heads, offs][:, None], 0.0)
    art = jnp.where(cols == heads[:, None], a_r[heads, offs][:, None], 0.0)
    grid = (R, N // TN)
    return pl.pallas_call(
        _trans_body,
        grid=grid,
        in_specs=[
            pl.BlockSpec((TN, in_dim), lambda r, t: (t, 0)),
            pl.BlockSpec((1, in_dim, ho), lambda r, t: (r, 0, 0)),
            pl.BlockSpec((ho, 8), lambda r, t: (0, 0)),
            pl.BlockSpec((ho, 8), lambda r, t: (0, 0)),
        ],
        out_specs=[
            pl.BlockSpec((1, TN, ho), lambda r, t: (r, t, 0)),
            pl.BlockSpec((1, TN, 16), lambda r, t: (r, t, 0)),
        ],
        out_shape=[
            jax.ShapeDtypeStruct((R, N, ho), jnp.float32),
            jax.ShapeDtypeStruct((R, N, 16), jnp.float32),
        ],
    )(h, W_rel, alt, art)


def _combine_body(h_ref, ws_ref, agg_ref, den_ref, exp_ref, o_ref, *, relu):
    s = jnp.dot(h_ref[...], ws_ref[...], preferred_element_type=jnp.float32)
    denf = jnp.dot(den_ref[...], exp_ref[...], preferred_element_type=jnp.float32)
    o = agg_ref[...] / (denf + 1e-9) + s
    o_ref[...] = jnp.maximum(o, 0.0) if relu else o


def _combine_call(h, W_self, agg, den8, relu):
    in_dim = h.shape[1]
    ho = W_self.shape[1]
    out = ho // H
    expand = (jnp.arange(8)[:, None] == (jnp.arange(ho) // out)[None, :]).astype(jnp.float32)
    grid = (N // TN,)
    return pl.pallas_call(
        functools.partial(_combine_body, relu=relu),
        grid=grid,
        in_specs=[
            pl.BlockSpec((TN, in_dim), lambda t: (t, 0)),
            pl.BlockSpec((in_dim, ho), lambda t: (0, 0)),
            pl.BlockSpec((TN, ho), lambda t: (t, 0)),
            pl.BlockSpec((TN, 8), lambda t: (t, 0)),
            pl.BlockSpec((8, ho), lambda t: (0, 0)),
        ],
        out_specs=pl.BlockSpec((TN, ho), lambda t: (t, 0)),
        out_shape=jax.ShapeDtypeStruct((N, ho), jnp.float32),
    )(h, W_self, agg, den8, expand)


def _edge_stage_xla(trans, lg, rn_src, rn_dst, dst):
    ho = trans.shape[2]
    out = ho // H
    lgf = lg.reshape(R * N, 16)
    g_src = lgf[rn_src]
    g_dst = lgf[rn_dst]
    e = g_src[:, 0:3] + g_dst[:, 8:11]
    e = jnp.where(e >= 0, e, 0.2 * e)
    ex = jnp.exp(e)
    den = jax.ops.segment_sum(ex, dst, num_segments=N)
    msg = trans.reshape(R * N, ho)[rn_src]
    w = jnp.repeat(ex, out, axis=1) * msg
    agg = jax.ops.segment_sum(w, dst, num_segments=N)
    den8 = jnp.pad(den, ((0, 0), (0, 5)))
    return agg, den8


def kernel(features, edge_index, edge_type, W_rel_0, a_l_0, a_r_0, W_self_0,
           W_rel_1, a_l_1, a_r_1, W_self_1, W_rel_2, a_l_2, a_r_2, W_self_2):
    src = edge_index[0]
    dst = edge_index[1]
    rn_src = edge_type * N + src
    rn_dst = edge_type * N + dst
    h = features
    layers = [
        (W_rel_0, a_l_0, a_r_0, W_self_0, True),
        (W_rel_1, a_l_1, a_r_1, W_self_1, True),
        (W_rel_2, a_l_2, a_r_2, W_self_2, False),
    ]
    for W_rel, a_l, a_r, W_self, relu in layers:
        trans, lg = _trans_call(h, W_rel, a_l, a_r)
        agg, den8 = _edge_stage_xla(trans, lg, rn_src, rn_dst, dst)
        h = _combine_call(h, W_self, agg, den8, relu)
    return h
